# Initial kernel scaffold; baseline (speedup 1.0000x reference)
import jax
import jax.numpy as jnp
from jax.experimental import pallas as pl


def kernel(feat_t, feat_tm1, agg_w, agg_b, proj_w, proj_b):
    raise NotImplementedError("write your pallas kernel here")


# speedup vs baseline: 8253.0069x; 8253.0069x over previous
import functools

import jax
import jax.numpy as jnp
from jax import lax
from jax.experimental import pallas as pl
from jax.experimental.pallas import tpu as pltpu

C = 64
H = 64
W = 64
HW = H * W
P2 = 9
D2 = 9
F32 = jnp.float32


def _wmask(v):
    w = lax.broadcasted_iota(jnp.int32, (1, HW), 1) % W
    return ((w + v >= 0) & (w + v < W)).astype(F32)


def _fshift(x, u, v):
    s = u * W + v
    n = x.shape[0]
    if s > 0:
        y = jnp.concatenate([x[:, s:], jnp.zeros((n, s), x.dtype)], axis=1)
    elif s < 0:
        y = jnp.concatenate([jnp.zeros((n, -s), x.dtype), x[:, :s]], axis=1)
    else:
        y = x
    if v != 0:
        y = y * _wmask(v)
    return y


def _boxsum(x):
    r = x + _fshift(x, 0, 1) + _fshift(x, 0, -1)
    return r + _fshift(r, 1, 0) + _fshift(r, -1, 0)


def _body(x_ref, wp_ref, pb_ref, aw_ref, ab_ref, out_ref, fms_ref):
    ft = x_ref[0:C, :]
    fm = x_ref[C:2 * C, :]

    for u in range(-2, 3):
        for v in range(-2, 3):
            fms_ref[(u + 2) * 5 + (v + 2)] = _fshift(fm, u, v)

    na = jnp.maximum(jnp.sqrt(_boxsum(jnp.sum(ft * ft, axis=0, keepdims=True))), 1e-12)
    nb = jnp.maximum(jnp.sqrt(_boxsum(jnp.sum(fm * fm, axis=0, keepdims=True))), 1e-12)

    hh = lax.broadcasted_iota(jnp.int32, (1, HW), 1) // W
    ww = lax.broadcasted_iota(jnp.int32, (1, HW), 1) % W

    sims = []
    valids = []
    for di in range(3):
        for dj in range(3):
            dy, dx = di - 1, dj - 1
            corr = jnp.sum(ft * fms_ref[(dy + 2) * 5 + (dx + 2)], axis=0, keepdims=True)
            raw = _boxsum(corr)
            nbs = jnp.maximum(_fshift(nb, dy, dx), 1e-12)
            valid = (hh + dy >= 0) & (hh + dy < H) & (ww + dx >= 0) & (ww + dx < W)
            sims.append(jnp.where(valid, raw / (na * nbs), 0.0))
            valids.append(valid)

    amaps = [jnp.zeros((1, HW), F32) for _ in range(D2)]
    cur = list(sims)
    for r in range(4):
        aw_r = aw_ref[r]
        best = cur[0]
        bidx = jnp.zeros((1, HW), jnp.int32)
        for dd in range(1, D2):
            cond = cur[dd] > best
            best = jnp.where(cond, cur[dd], best)
            bidx = jnp.where(cond, dd, bidx)
        for dd in range(D2):
            amaps[dd] = amaps[dd] + jnp.where((bidx == dd) & valids[dd], aw_r, 0.0)
        cur = [jnp.where(bidx == dd, -5.0, cur[dd]) for dd in range(D2)]
    ab = ab_ref[0]
    a_b = [jnp.broadcast_to(amaps[dd], (C, HW)) for dd in range(D2)]

    acc = jnp.zeros((C, HW), F32)
    for pp in range(P2):
        pi, pj = pp // 3, pp % 3
        wp_pp = jnp.dot(wp_ref[pp * C:(pp + 1) * C, :], x_ref[...],
                        preferred_element_type=F32) + pb_ref[pp * C:(pp + 1) * C, :]
        b_pp = jnp.zeros((C, HW), F32)
        for di in range(3):
            for dj in range(3):
                dd = di * 3 + dj
                sel = (di - 1 + pi - 1 + 2) * 5 + (dj - 1 + pj - 1 + 2)
                b_pp = b_pp + a_b[dd] * fms_ref[sel]
        acc = acc + wp_pp * (b_pp + ab)
    out_ref[...] = acc


@functools.partial(jax.jit, static_argnames=())
def kernel(feat_t, feat_tm1, agg_w, agg_b, proj_w, proj_b):
    x = jnp.concatenate([feat_t.reshape(C, HW), feat_tm1.reshape(C, HW)], axis=0)
    wp2 = proj_w.reshape(C, P2, 2 * C).transpose(1, 0, 2).reshape(P2 * C, 2 * C)
    pb2 = proj_b.reshape(C, P2).T.reshape(P2 * C, 1)
    aw = agg_w.reshape(4)
    ab = agg_b.reshape(1)
    out = pl.pallas_call(
        _body,
        out_shape=jax.ShapeDtypeStruct((C, HW), F32),
        in_specs=[
            pl.BlockSpec(memory_space=pltpu.VMEM),
            pl.BlockSpec(memory_space=pltpu.VMEM),
            pl.BlockSpec(memory_space=pltpu.VMEM),
            pl.BlockSpec(memory_space=pltpu.SMEM),
            pl.BlockSpec(memory_space=pltpu.SMEM),
        ],
        scratch_shapes=[pltpu.VMEM((25, C, HW), F32)],
    )(x, wp2, pb2, aw, ab)
    return out.reshape(1, C, H, W)
